# baseline (speedup 1.0000x reference)
import functools

import jax
import jax.numpy as jnp
from jax import lax
from jax.experimental import pallas as pl
from jax.experimental.pallas import tpu as pltpu
from jax.experimental.pallas import tpu_sc as plsc

B = 8
N = 1000
C = 81
K = 100
MIN_CONF = 0.7
NMS_THR = 0.3
N1 = 1024
NCH = N1 // 16
NC = 96
BIG = 2**30


def _dense_body(rois_ref, probs_ref, flat_ref, osc_ref, oy1_ref, ox1_ref,
                oy2_ref, ox2_ref, ocl_ref):
    probs = probs_ref[0]
    rois = rois_ref[0]
    flat = flat_ref[0]

    m = jnp.max(probs, axis=1, keepdims=True)
    iota_c = lax.broadcasted_iota(jnp.int32, (N, C), 1)
    cid = jnp.min(jnp.where(probs == m, iota_c, C), axis=1, keepdims=True)

    iota_f = lax.broadcasted_iota(jnp.int32, (N, 4 * C), 1)
    masked = jnp.where(lax.shift_right_logical(iota_f, 2) == cid, flat, 0.0)
    iota_j = lax.broadcasted_iota(jnp.int32, (4 * C, 4), 0)
    iota_k = lax.broadcasted_iota(jnp.int32, (4 * C, 4), 1)
    sel = (lax.rem(iota_j, 4) == iota_k).astype(jnp.float32)
    dsel = lax.dot(masked, sel, precision=lax.Precision.HIGHEST)
    dy = dsel[:, 0:1] * 0.1
    dx = dsel[:, 1:2] * 0.1
    dh = dsel[:, 2:3] * 0.2
    dw = dsel[:, 3:4] * 0.2

    ry1 = rois[:, 0:1]
    rx1 = rois[:, 1:2]
    ry2 = rois[:, 2:3]
    rx2 = rois[:, 3:4]
    h = ry2 - ry1
    w = rx2 - rx1
    cy = ry1 + 0.5 * h
    cx = rx1 + 0.5 * w
    cy = cy + dy * h
    cx = cx + dx * w
    h = h * jnp.exp(dh)
    w = w * jnp.exp(dw)
    y1 = jnp.clip(cy - 0.5 * h, 0.0, 1.0)
    x1 = jnp.clip(cx - 0.5 * w, 0.0, 1.0)
    y2 = jnp.clip(cy + 0.5 * h, 0.0, 1.0)
    x2 = jnp.clip(cx + 0.5 * w, 0.0, 1.0)

    valid = (cid > 0) & (m >= MIN_CONF)
    sc = jnp.where(valid, m, -1.0)
    clsf = cid.astype(jnp.float32)

    padn = jnp.full((N1 - N,), -1.0, jnp.float32)
    padz = jnp.zeros((N1 - N,), jnp.float32)
    osc_ref[...] = jnp.concatenate([jnp.reshape(sc, (N,)), padn])
    oy1_ref[...] = jnp.concatenate([jnp.reshape(y1, (N,)), padz])
    ox1_ref[...] = jnp.concatenate([jnp.reshape(x1, (N,)), padz])
    oy2_ref[...] = jnp.concatenate([jnp.reshape(y2, (N,)), padz])
    ox2_ref[...] = jnp.concatenate([jnp.reshape(x2, (N,)), padz])
    ocl_ref[...] = jnp.concatenate([jnp.reshape(clsf, (N,)), padz])


def _sc_nms_body(hsc, hy1, hx1, hy2, hx2, hcl, out_hbm,
                 dsc, dy1, dx1, dy2, dx2, dcl,
                 ccl, rnk, lst,
                 bsc, by1, bx1, by2, bx2, bcl, bidx,
                 counts, bases, wbase, cmax, ctie, obuf):
    wid = lax.axis_index("s") * 2 + lax.axis_index("c")

    @pl.when(wid < B)
    def _():
        base = wid * N1
        pltpu.sync_copy(hsc.at[pl.ds(base, N1)], dsc)
        pltpu.sync_copy(hy1.at[pl.ds(base, N1)], dy1)
        pltpu.sync_copy(hx1.at[pl.ds(base, N1)], dx1)
        pltpu.sync_copy(hy2.at[pl.ds(base, N1)], dy2)
        pltpu.sync_copy(hx2.at[pl.ds(base, N1)], dx2)
        pltpu.sync_copy(hcl.at[pl.ds(base, N1)], dcl)

        iota = lax.iota(jnp.int32, 16)
        zeros16i = jnp.zeros((16,), jnp.int32)
        zeros16f = jnp.zeros((16,), jnp.float32)

        for t in range(NC // 16):
            counts[pl.ds(t * 16, 16)] = zeros16i

        def body1(q, c):
            b16 = q * 16
            cls = dcl[pl.ds(b16, 16)].astype(jnp.int32)
            rank = jnp.zeros((16,), jnp.int32)
            after = jnp.zeros((16,), jnp.int32)
            one = jnp.full((16,), 1, jnp.int32)
            zero = jnp.zeros((16,), jnp.int32)
            for j in range(1, 16):
                dn = cls[jnp.maximum(iota - j, 0)]
                up = cls[jnp.minimum(iota + j, 15)]
                rank = rank + jnp.where((dn == cls) & (iota >= j), one, zero)
                after = after + jnp.where((up == cls) & (iota < 16 - j),
                                          one, zero)
            is_last = after == 0
            ccl[pl.ds(b16, 16)] = cls
            rnk[pl.ds(b16, 16)] = rank
            lst[pl.ds(b16, 16)] = jnp.where(is_last, one, zero)
            plsc.addupdate_scatter(counts, [cls], rank + 1, mask=is_last)
            return c

        lax.fori_loop(0, NCH, body1, 0)

        run = jnp.int32(0)
        for t in range(NC // 16):
            v = counts[pl.ds(t * 16, 16)]
            cs = plsc.cumsum(v)
            ex = cs - v + run
            bases[pl.ds(t * 16, 16)] = ex
            wbase[pl.ds(t * 16, 16)] = ex
            run = run + jnp.max(cs)

        def body3(q, c):
            b16 = q * 16
            cls = ccl[pl.ds(b16, 16)]
            rank = rnk[pl.ds(b16, 16)]
            is_last = lst[pl.ds(b16, 16)] == 1
            pos = plsc.load_gather(wbase, [cls]) + rank
            plsc.store_scatter(bsc, [pos], dsc[pl.ds(b16, 16)])
            plsc.store_scatter(by1, [pos], dy1[pl.ds(b16, 16)])
            plsc.store_scatter(bx1, [pos], dx1[pl.ds(b16, 16)])
            plsc.store_scatter(by2, [pos], dy2[pl.ds(b16, 16)])
            plsc.store_scatter(bx2, [pos], dx2[pl.ds(b16, 16)])
            plsc.store_scatter(bcl, [pos], dcl[pl.ds(b16, 16)])
            plsc.store_scatter(bidx, [pos], b16 + iota)
            plsc.addupdate_scatter(wbase, [cls], rank + 1, mask=is_last)
            return c

        lax.fori_loop(0, NCH, body3, 0)

        def body4(q, c):
            b16 = q * 16
            s = bsc[pl.ds(b16, 16)]
            mc = jnp.max(s)
            bi = bidx[pl.ds(b16, 16)]
            tc = jnp.min(jnp.where(s == mc, bi, BIG))
            lane0 = iota == 0
            qv = jnp.full((16,), q, jnp.int32)
            plsc.store_scatter(cmax, [qv], jnp.full((16,), mc, jnp.float32),
                               mask=lane0)
            plsc.store_scatter(ctie, [qv], jnp.full((16,), tc, jnp.int32),
                               mask=lane0)
            return c

        lax.fori_loop(0, NCH, body4, 0)

        def bodyz(q, c):
            obuf[pl.ds(q * 16, 16)] = zeros16f
            return c

        lax.fori_loop(0, K, bodyz, 0)

        def global_max():
            c0 = cmax[pl.ds(0, 16)]
            c1 = cmax[pl.ds(16, 16)]
            c2 = cmax[pl.ds(32, 16)]
            c3 = cmax[pl.ds(48, 16)]
            return jnp.max(jnp.maximum(jnp.maximum(c0, c1),
                                       jnp.maximum(c2, c3)))

        def wcond(st):
            k, m = st
            return (k < K) & (m > 0.0)

        def wbody(st):
            k, m = st
            c0 = cmax[pl.ds(0, 16)]
            c1 = cmax[pl.ds(16, 16)]
            c2 = cmax[pl.ds(32, 16)]
            c3 = cmax[pl.ds(48, 16)]
            t0 = ctie[pl.ds(0, 16)]
            t1 = ctie[pl.ds(16, 16)]
            t2 = ctie[pl.ds(32, 16)]
            t3 = ctie[pl.ds(48, 16)]
            big = jnp.full((16,), BIG, jnp.int32)
            a0 = jnp.where(c0 == m, t0, big)
            a1 = jnp.where(c1 == m, t1, big)
            a2 = jnp.where(c2 == m, t2, big)
            a3 = jnp.where(c3 == m, t3, big)
            tmin = jnp.min(jnp.minimum(jnp.minimum(a0, a1),
                                       jnp.minimum(a2, a3)))
            q0 = jnp.where((c0 == m) & (t0 == tmin), iota, big)
            q1 = jnp.where((c1 == m) & (t1 == tmin), iota + 16, big)
            q2 = jnp.where((c2 == m) & (t2 == tmin), iota + 32, big)
            q3 = jnp.where((c3 == m) & (t3 == tmin), iota + 48, big)
            qs = jnp.min(jnp.minimum(jnp.minimum(q0, q1),
                                     jnp.minimum(q2, q3)))
            qs = jnp.clip(qs, 0, NCH - 1)
            b16 = qs * 16
            s = bsc[pl.ds(b16, 16)]
            bi = bidx[pl.ds(b16, 16)]
            ps = jnp.min(jnp.where((s == m) & (bi == tmin), iota, 15))
            g = b16 + ps
            gv = jnp.full((16,), g, jnp.int32)
            lane0 = iota == 0
            vy1 = plsc.load_gather(by1, [gv])
            vx1 = plsc.load_gather(bx1, [gv])
            vy2 = plsc.load_gather(by2, [gv])
            vx2 = plsc.load_gather(bx2, [gv])
            vcl = plsc.load_gather(bcl, [gv])
            soff = vcl * 10.0
            sny1 = vy1 + soff
            snx1 = vx1 + soff
            sny2 = vy2 + soff
            snx2 = vx2 + soff
            sarea = (sny2 - sny1) * (snx2 - snx1)
            out16 = jnp.where(iota == 0, vy1, 0.0)
            out16 = jnp.where(iota == 1, vx1, out16)
            out16 = jnp.where(iota == 2, vy2, out16)
            out16 = jnp.where(iota == 3, vx2, out16)
            out16 = jnp.where(iota == 4, vcl, out16)
            out16 = jnp.where(iota == 5, jnp.full((16,), m), out16)
            obuf[pl.ds(k * 16, 16)] = out16
            civ = jnp.clip(vcl.astype(jnp.int32), 0, NC - 1)
            a = jnp.max(plsc.load_gather(bases, [civ]))
            bnd = a + jnp.max(plsc.load_gather(counts, [civ]))
            plsc.store_scatter(bsc, [gv],
                               jnp.full((16,), -1.0, jnp.float32),
                               mask=lane0)
            qa = jnp.clip(lax.div(a, 16), 0, NCH - 1)
            qb = jnp.clip(lax.div(bnd + 15, 16), 0, NCH)

            def sbody(qq, c):
                o = qq * 16
                gi = o + iota
                sv = bsc[pl.ds(o, 16)]
                voff = bcl[pl.ds(o, 16)] * 10.0
                vny1 = by1[pl.ds(o, 16)] + voff
                vnx1 = bx1[pl.ds(o, 16)] + voff
                vny2 = by2[pl.ds(o, 16)] + voff
                vnx2 = bx2[pl.ds(o, 16)] + voff
                varea = (vny2 - vny1) * (vnx2 - vnx1)
                yy1 = jnp.maximum(vny1, sny1)
                xx1 = jnp.maximum(vnx1, snx1)
                yy2 = jnp.minimum(vny2, sny2)
                xx2 = jnp.minimum(vnx2, snx2)
                inter = (jnp.maximum(yy2 - yy1, 0.0)
                         * jnp.maximum(xx2 - xx1, 0.0))
                union = sarea + varea - inter
                iou = inter / jnp.maximum(union, 1e-8)
                kill = ((gi >= a) & (gi < bnd) & (gi != g)
                        & (iou > NMS_THR))
                s2 = jnp.where(kill, -1.0, sv)
                bsc[pl.ds(o, 16)] = s2
                mc = jnp.max(s2)
                tc = jnp.min(jnp.where(s2 == mc, bidx[pl.ds(o, 16)], BIG))
                qv = jnp.full((16,), qq, jnp.int32)
                plsc.store_scatter(cmax, [qv],
                                   jnp.full((16,), mc, jnp.float32),
                                   mask=lane0)
                plsc.store_scatter(ctie, [qv],
                                   jnp.full((16,), tc, jnp.int32),
                                   mask=lane0)
                return c

            lax.fori_loop(qa, qb, sbody, 0)
            return (k + 1, global_max())

        lax.while_loop(wcond, wbody, (jnp.int32(0), global_max()))
        pltpu.sync_copy(obuf, out_hbm.at[wid])


def _sc_nms(data):
    mesh = plsc.VectorSubcoreMesh(core_axis_name="c", subcore_axis_name="s")
    f32 = jnp.float32
    i32 = jnp.int32
    run = functools.partial(
        pl.kernel,
        mesh=mesh,
        compiler_params=pltpu.CompilerParams(needs_layout_passes=False),
        out_type=jax.ShapeDtypeStruct((B, K * 16), f32),
        scratch_types=[
            pltpu.VMEM((N1,), f32),
            pltpu.VMEM((N1,), f32),
            pltpu.VMEM((N1,), f32),
            pltpu.VMEM((N1,), f32),
            pltpu.VMEM((N1,), f32),
            pltpu.VMEM((N1,), f32),
            pltpu.VMEM((N1,), i32),
            pltpu.VMEM((N1,), i32),
            pltpu.VMEM((N1,), i32),
            pltpu.VMEM((N1,), f32),
            pltpu.VMEM((N1,), f32),
            pltpu.VMEM((N1,), f32),
            pltpu.VMEM((N1,), f32),
            pltpu.VMEM((N1,), f32),
            pltpu.VMEM((N1,), f32),
            pltpu.VMEM((N1,), i32),
            pltpu.VMEM((NC,), i32),
            pltpu.VMEM((NC,), i32),
            pltpu.VMEM((NC,), i32),
            pltpu.VMEM((NCH,), f32),
            pltpu.VMEM((NCH,), i32),
            pltpu.VMEM((K * 16,), f32),
        ],
    )(_sc_nms_body)
    return run(*data)


def kernel(rois, mrcnn_class, mrcnn_bbox):
    flat = mrcnn_bbox.reshape(B, N, 4 * C)
    data = pl.pallas_call(
        _dense_body,
        grid=(B,),
        in_specs=[
            pl.BlockSpec((1, N, 4), lambda b: (b, 0, 0)),
            pl.BlockSpec((1, N, C), lambda b: (b, 0, 0)),
            pl.BlockSpec((1, N, 4 * C), lambda b: (b, 0, 0)),
        ],
        out_specs=[pl.BlockSpec((N1,), lambda b: (b,))] * 6,
        out_shape=[jax.ShapeDtypeStruct((B * N1,), jnp.float32)] * 6,
    )(rois, mrcnn_class, flat)
    out = _sc_nms(data)
    det = out.reshape(B, K, 16)[:, :, :6]
    return det


# speedup vs baseline: 1.8285x; 1.8285x over previous
import functools

import jax
import jax.numpy as jnp
from jax import lax
from jax.experimental import pallas as pl
from jax.experimental.pallas import tpu as pltpu
from jax.experimental.pallas import tpu_sc as plsc

B = 8
N = 1000
C = 81
K = 100
MIN_CONF = 0.7
NMS_THR = 0.3
N1 = 1024
NCH = N1 // 16
NC = 96
BIG = 2**30


def _dense_body(rois_ref, probs_ref, flat_ref, osc_ref, oy1_ref, ox1_ref,
                oy2_ref, ox2_ref, ocl_ref):
    for b in range(B):
        _dense_one(rois_ref[b], probs_ref[:, b, :], flat_ref[b],
                   b, osc_ref, oy1_ref, ox1_ref, oy2_ref, ox2_ref, ocl_ref)


def _dense_one(rois, probs, flat4, b, osc_ref, oy1_ref, ox1_ref,
               oy2_ref, ox2_ref, ocl_ref):
    m = jnp.max(probs, axis=0, keepdims=True)
    iota_c = lax.broadcasted_iota(jnp.int32, (C, N), 0)
    cid = jnp.min(jnp.where(probs == m, iota_c, C), axis=0, keepdims=True)

    maskf = (iota_c == cid).astype(jnp.float32)
    s = flat4 * maskf[:, None, :]
    while s.shape[0] > 1:
        half = s.shape[0] // 2
        even = s[:half] + s[half:2 * half]
        if s.shape[0] % 2:
            even = jnp.concatenate(
                [even[:1] + s[2 * half:], even[1:]], axis=0)
        s = even
    dsel = s[0]
    dy = dsel[0:1] * 0.1
    dx = dsel[1:2] * 0.1
    dh = dsel[2:3] * 0.2
    dw = dsel[3:4] * 0.2

    ry1 = rois[0:1]
    rx1 = rois[1:2]
    ry2 = rois[2:3]
    rx2 = rois[3:4]
    h = ry2 - ry1
    w = rx2 - rx1
    cy = ry1 + 0.5 * h
    cx = rx1 + 0.5 * w
    cy = cy + dy * h
    cx = cx + dx * w
    h = h * jnp.exp(dh)
    w = w * jnp.exp(dw)
    y1 = jnp.clip(cy - 0.5 * h, 0.0, 1.0)
    x1 = jnp.clip(cx - 0.5 * w, 0.0, 1.0)
    y2 = jnp.clip(cy + 0.5 * h, 0.0, 1.0)
    x2 = jnp.clip(cx + 0.5 * w, 0.0, 1.0)

    valid = (cid > 0) & (m >= MIN_CONF)
    sc = jnp.where(valid, m, -1.0)
    clsf = cid.astype(jnp.float32)

    padn = jnp.full((N1 - N,), -1.0, jnp.float32)
    padz = jnp.zeros((N1 - N,), jnp.float32)
    o = b * N1
    osc_ref[pl.ds(o, N1)] = jnp.concatenate([jnp.reshape(sc, (N,)), padn])
    oy1_ref[pl.ds(o, N1)] = jnp.concatenate([jnp.reshape(y1, (N,)), padz])
    ox1_ref[pl.ds(o, N1)] = jnp.concatenate([jnp.reshape(x1, (N,)), padz])
    oy2_ref[pl.ds(o, N1)] = jnp.concatenate([jnp.reshape(y2, (N,)), padz])
    ox2_ref[pl.ds(o, N1)] = jnp.concatenate([jnp.reshape(x2, (N,)), padz])
    ocl_ref[pl.ds(o, N1)] = jnp.concatenate([jnp.reshape(clsf, (N,)), padz])


def _sc_nms_body(hsc, hy1, hx1, hy2, hx2, hcl, out_hbm,
                 dsc, dy1, dx1, dy2, dx2, dcl,
                 ccl, rnk, lst,
                 bsc, by1, bx1, by2, bx2, bcl, bidx,
                 counts, bases, wbase, cmax, ctie, obuf):
    wid = lax.axis_index("s") * 2 + lax.axis_index("c")

    @pl.when(wid < B)
    def _():
        base = wid * N1
        pltpu.sync_copy(hsc.at[pl.ds(base, N1)], dsc)
        pltpu.sync_copy(hy1.at[pl.ds(base, N1)], dy1)
        pltpu.sync_copy(hx1.at[pl.ds(base, N1)], dx1)
        pltpu.sync_copy(hy2.at[pl.ds(base, N1)], dy2)
        pltpu.sync_copy(hx2.at[pl.ds(base, N1)], dx2)
        pltpu.sync_copy(hcl.at[pl.ds(base, N1)], dcl)

        iota = lax.iota(jnp.int32, 16)
        zeros16i = jnp.zeros((16,), jnp.int32)
        zeros16f = jnp.zeros((16,), jnp.float32)

        for t in range(NC // 16):
            counts[pl.ds(t * 16, 16)] = zeros16i

        def body1(q, c):
            b16 = q * 16
            cls = dcl[pl.ds(b16, 16)].astype(jnp.int32)
            cnt, is_last = plsc.scan_count(cls)
            rank = cnt - 1
            ccl[pl.ds(b16, 16)] = cls
            rnk[pl.ds(b16, 16)] = rank
            lst[pl.ds(b16, 16)] = jnp.where(is_last,
                                            jnp.full((16,), 1, jnp.int32),
                                            jnp.zeros((16,), jnp.int32))
            plsc.addupdate_scatter(counts, [cls], cnt, mask=is_last)
            return c

        lax.fori_loop(0, NCH, body1, 0)

        run = jnp.int32(0)
        for t in range(NC // 16):
            v = counts[pl.ds(t * 16, 16)]
            cs = plsc.cumsum(v)
            ex = cs - v + run
            bases[pl.ds(t * 16, 16)] = ex
            wbase[pl.ds(t * 16, 16)] = ex
            run = run + jnp.max(cs)

        def body3(q, c):
            b16 = q * 16
            cls = ccl[pl.ds(b16, 16)]
            rank = rnk[pl.ds(b16, 16)]
            is_last = lst[pl.ds(b16, 16)] == 1
            pos = plsc.load_gather(wbase, [cls]) + rank
            plsc.store_scatter(bsc, [pos], dsc[pl.ds(b16, 16)])
            plsc.store_scatter(by1, [pos], dy1[pl.ds(b16, 16)])
            plsc.store_scatter(bx1, [pos], dx1[pl.ds(b16, 16)])
            plsc.store_scatter(by2, [pos], dy2[pl.ds(b16, 16)])
            plsc.store_scatter(bx2, [pos], dx2[pl.ds(b16, 16)])
            plsc.store_scatter(bcl, [pos], dcl[pl.ds(b16, 16)])
            plsc.store_scatter(bidx, [pos], b16 + iota)
            plsc.addupdate_scatter(wbase, [cls], rank + 1, mask=is_last)
            return c

        lax.fori_loop(0, NCH, body3, 0)

        def body4(q, c):
            b16 = q * 16
            s = bsc[pl.ds(b16, 16)]
            mc = jnp.max(s)
            bi = bidx[pl.ds(b16, 16)]
            tc = jnp.min(jnp.where(s == mc, bi, BIG))
            lane0 = iota == 0
            qv = jnp.full((16,), q, jnp.int32)
            plsc.store_scatter(cmax, [qv], jnp.full((16,), mc, jnp.float32),
                               mask=lane0)
            plsc.store_scatter(ctie, [qv], jnp.full((16,), tc, jnp.int32),
                               mask=lane0)
            return c

        lax.fori_loop(0, NCH, body4, 0)

        def bodyz(q, c):
            obuf[pl.ds(q * 16, 16)] = zeros16f
            return c

        lax.fori_loop(0, K, bodyz, 0)

        def global_max():
            c0 = cmax[pl.ds(0, 16)]
            c1 = cmax[pl.ds(16, 16)]
            c2 = cmax[pl.ds(32, 16)]
            c3 = cmax[pl.ds(48, 16)]
            return jnp.max(jnp.maximum(jnp.maximum(c0, c1),
                                       jnp.maximum(c2, c3)))

        def wcond(st):
            k, m = st
            return (k < K) & (m > 0.0)

        def wbody(st):
            k, m = st
            c0 = cmax[pl.ds(0, 16)]
            c1 = cmax[pl.ds(16, 16)]
            c2 = cmax[pl.ds(32, 16)]
            c3 = cmax[pl.ds(48, 16)]
            t0 = ctie[pl.ds(0, 16)]
            t1 = ctie[pl.ds(16, 16)]
            t2 = ctie[pl.ds(32, 16)]
            t3 = ctie[pl.ds(48, 16)]
            big = jnp.full((16,), BIG, jnp.int32)
            a0 = jnp.where(c0 == m, t0, big)
            a1 = jnp.where(c1 == m, t1, big)
            a2 = jnp.where(c2 == m, t2, big)
            a3 = jnp.where(c3 == m, t3, big)
            tmin = jnp.min(jnp.minimum(jnp.minimum(a0, a1),
                                       jnp.minimum(a2, a3)))
            q0 = jnp.where((c0 == m) & (t0 == tmin), iota, big)
            q1 = jnp.where((c1 == m) & (t1 == tmin), iota + 16, big)
            q2 = jnp.where((c2 == m) & (t2 == tmin), iota + 32, big)
            q3 = jnp.where((c3 == m) & (t3 == tmin), iota + 48, big)
            qs = jnp.min(jnp.minimum(jnp.minimum(q0, q1),
                                     jnp.minimum(q2, q3)))
            qs = jnp.clip(qs, 0, NCH - 1)
            b16 = qs * 16
            s = bsc[pl.ds(b16, 16)]
            bi = bidx[pl.ds(b16, 16)]
            ps = jnp.min(jnp.where((s == m) & (bi == tmin), iota, 15))
            g = b16 + ps
            gv = jnp.full((16,), g, jnp.int32)
            lane0 = iota == 0
            vy1 = plsc.load_gather(by1, [gv])
            vx1 = plsc.load_gather(bx1, [gv])
            vy2 = plsc.load_gather(by2, [gv])
            vx2 = plsc.load_gather(bx2, [gv])
            vcl = plsc.load_gather(bcl, [gv])
            soff = vcl * 10.0
            sny1 = vy1 + soff
            snx1 = vx1 + soff
            sny2 = vy2 + soff
            snx2 = vx2 + soff
            sarea = (sny2 - sny1) * (snx2 - snx1)
            out16 = jnp.where(iota == 0, vy1, 0.0)
            out16 = jnp.where(iota == 1, vx1, out16)
            out16 = jnp.where(iota == 2, vy2, out16)
            out16 = jnp.where(iota == 3, vx2, out16)
            out16 = jnp.where(iota == 4, vcl, out16)
            out16 = jnp.where(iota == 5, jnp.full((16,), m), out16)
            obuf[pl.ds(k * 16, 16)] = out16
            civ = jnp.clip(vcl.astype(jnp.int32), 0, NC - 1)
            a = jnp.max(plsc.load_gather(bases, [civ]))
            bnd = a + jnp.max(plsc.load_gather(counts, [civ]))
            plsc.store_scatter(bsc, [gv],
                               jnp.full((16,), -1.0, jnp.float32),
                               mask=lane0)
            qa = jnp.clip(lax.div(a, 16), 0, NCH - 1)
            qb = jnp.clip(lax.div(bnd + 15, 16), 0, NCH)

            def sbody(qq, c):
                o = qq * 16
                gi = o + iota
                sv = bsc[pl.ds(o, 16)]
                voff = bcl[pl.ds(o, 16)] * 10.0
                vny1 = by1[pl.ds(o, 16)] + voff
                vnx1 = bx1[pl.ds(o, 16)] + voff
                vny2 = by2[pl.ds(o, 16)] + voff
                vnx2 = bx2[pl.ds(o, 16)] + voff
                varea = (vny2 - vny1) * (vnx2 - vnx1)
                yy1 = jnp.maximum(vny1, sny1)
                xx1 = jnp.maximum(vnx1, snx1)
                yy2 = jnp.minimum(vny2, sny2)
                xx2 = jnp.minimum(vnx2, snx2)
                inter = (jnp.maximum(yy2 - yy1, 0.0)
                         * jnp.maximum(xx2 - xx1, 0.0))
                union = sarea + varea - inter
                iou = inter / jnp.maximum(union, 1e-8)
                kill = ((gi >= a) & (gi < bnd) & (gi != g)
                        & (iou > NMS_THR))
                s2 = jnp.where(kill, -1.0, sv)
                bsc[pl.ds(o, 16)] = s2
                mc = jnp.max(s2)
                tc = jnp.min(jnp.where(s2 == mc, bidx[pl.ds(o, 16)], BIG))
                qv = jnp.full((16,), qq, jnp.int32)
                plsc.store_scatter(cmax, [qv],
                                   jnp.full((16,), mc, jnp.float32),
                                   mask=lane0)
                plsc.store_scatter(ctie, [qv],
                                   jnp.full((16,), tc, jnp.int32),
                                   mask=lane0)
                return c

            lax.fori_loop(qa, qb, sbody, 0)
            return (k + 1, global_max())

        lax.while_loop(wcond, wbody, (jnp.int32(0), global_max()))
        pltpu.sync_copy(obuf, out_hbm.at[wid])


def _sc_nms(data):
    mesh = plsc.VectorSubcoreMesh(core_axis_name="c", subcore_axis_name="s")
    f32 = jnp.float32
    i32 = jnp.int32
    run = functools.partial(
        pl.kernel,
        mesh=mesh,
        compiler_params=pltpu.CompilerParams(needs_layout_passes=False),
        out_type=jax.ShapeDtypeStruct((B, K * 16), f32),
        scratch_types=[
            pltpu.VMEM((N1,), f32),
            pltpu.VMEM((N1,), f32),
            pltpu.VMEM((N1,), f32),
            pltpu.VMEM((N1,), f32),
            pltpu.VMEM((N1,), f32),
            pltpu.VMEM((N1,), f32),
            pltpu.VMEM((N1,), i32),
            pltpu.VMEM((N1,), i32),
            pltpu.VMEM((N1,), i32),
            pltpu.VMEM((N1,), f32),
            pltpu.VMEM((N1,), f32),
            pltpu.VMEM((N1,), f32),
            pltpu.VMEM((N1,), f32),
            pltpu.VMEM((N1,), f32),
            pltpu.VMEM((N1,), f32),
            pltpu.VMEM((N1,), i32),
            pltpu.VMEM((NC,), i32),
            pltpu.VMEM((NC,), i32),
            pltpu.VMEM((NC,), i32),
            pltpu.VMEM((NCH,), f32),
            pltpu.VMEM((NCH,), i32),
            pltpu.VMEM((K * 16,), f32),
        ],
    )(_sc_nms_body)
    return run(*data)


def kernel(rois, mrcnn_class, mrcnn_bbox):
    rois_t = jnp.transpose(rois, (0, 2, 1))
    probs_t = jnp.transpose(mrcnn_class, (2, 0, 1))
    flat_t = jnp.transpose(mrcnn_bbox, (0, 2, 3, 1))
    data = pl.pallas_call(
        _dense_body,
        out_shape=[jax.ShapeDtypeStruct((B * N1,), jnp.float32)] * 6,
    )(rois_t, probs_t, flat_t)
    out = _sc_nms(data)
    det = out.reshape(B, K, 16)[:, :, :6]
    return det


# speedup vs baseline: 1.9051x; 1.0419x over previous
import functools

import jax
import jax.numpy as jnp
from jax import lax
from jax.experimental import pallas as pl
from jax.experimental.pallas import tpu as pltpu
from jax.experimental.pallas import tpu_sc as plsc

B = 8
N = 1000
C = 81
K = 100
MIN_CONF = 0.7
NMS_THR = 0.3
N1 = 1024
NCH = N1 // 16
NC = 96
BIG = 2**30


def _dense_body(rois_ref, probs_ref, flat_ref, osc_ref, oy1_ref, ox1_ref,
                oy2_ref, ox2_ref, ocl_ref):
    for b in range(B):
        _dense_one(rois_ref[b], probs_ref[:, b, :], flat_ref[b],
                   b, osc_ref, oy1_ref, ox1_ref, oy2_ref, ox2_ref, ocl_ref)


def _dense_one(rois, probs, flat4, b, osc_ref, oy1_ref, ox1_ref,
               oy2_ref, ox2_ref, ocl_ref):
    m = jnp.max(probs, axis=0, keepdims=True)
    iota_c = lax.broadcasted_iota(jnp.int32, (C, N), 0)
    cid = jnp.min(jnp.where(probs == m, iota_c, C), axis=0, keepdims=True)

    maskf = (iota_c == cid).astype(jnp.float32)
    s = flat4 * maskf[:, None, :]
    while s.shape[0] > 1:
        half = s.shape[0] // 2
        even = s[:half] + s[half:2 * half]
        if s.shape[0] % 2:
            even = jnp.concatenate(
                [even[:1] + s[2 * half:], even[1:]], axis=0)
        s = even
    dsel = s[0]
    dy = dsel[0:1] * 0.1
    dx = dsel[1:2] * 0.1
    dh = dsel[2:3] * 0.2
    dw = dsel[3:4] * 0.2

    ry1 = rois[0:1]
    rx1 = rois[1:2]
    ry2 = rois[2:3]
    rx2 = rois[3:4]
    h = ry2 - ry1
    w = rx2 - rx1
    cy = ry1 + 0.5 * h
    cx = rx1 + 0.5 * w
    cy = cy + dy * h
    cx = cx + dx * w
    h = h * jnp.exp(dh)
    w = w * jnp.exp(dw)
    y1 = jnp.clip(cy - 0.5 * h, 0.0, 1.0)
    x1 = jnp.clip(cx - 0.5 * w, 0.0, 1.0)
    y2 = jnp.clip(cy + 0.5 * h, 0.0, 1.0)
    x2 = jnp.clip(cx + 0.5 * w, 0.0, 1.0)

    valid = (cid > 0) & (m >= MIN_CONF)
    sc = jnp.where(valid, m, -1.0)
    clsf = cid.astype(jnp.float32)

    padn = jnp.full((N1 - N,), -1.0, jnp.float32)
    padz = jnp.zeros((N1 - N,), jnp.float32)
    o = b * N1
    osc_ref[pl.ds(o, N1)] = jnp.concatenate([jnp.reshape(sc, (N,)), padn])
    oy1_ref[pl.ds(o, N1)] = jnp.concatenate([jnp.reshape(y1, (N,)), padz])
    ox1_ref[pl.ds(o, N1)] = jnp.concatenate([jnp.reshape(x1, (N,)), padz])
    oy2_ref[pl.ds(o, N1)] = jnp.concatenate([jnp.reshape(y2, (N,)), padz])
    ox2_ref[pl.ds(o, N1)] = jnp.concatenate([jnp.reshape(x2, (N,)), padz])
    ocl_ref[pl.ds(o, N1)] = jnp.concatenate([jnp.reshape(clsf, (N,)), padz])


def _sc_nms_body(hsc, hy1, hx1, hy2, hx2, hcl, out_hbm,
                 dsc, dy1, dx1, dy2, dx2, dcl,
                 ccl, rnk, lst,
                 bsc, by1, bx1, by2, bx2, bcl, bidx,
                 counts, bases, wbase, cmax, ctie, obuf, dsem):
    wid = lax.axis_index("s") * 2 + lax.axis_index("c")

    @pl.when(wid < B)
    def _():
        base = wid * N1
        cps = [
            pltpu.make_async_copy(hsc.at[pl.ds(base, N1)], dsc, dsem),
            pltpu.make_async_copy(hy1.at[pl.ds(base, N1)], dy1, dsem),
            pltpu.make_async_copy(hx1.at[pl.ds(base, N1)], dx1, dsem),
            pltpu.make_async_copy(hy2.at[pl.ds(base, N1)], dy2, dsem),
            pltpu.make_async_copy(hx2.at[pl.ds(base, N1)], dx2, dsem),
            pltpu.make_async_copy(hcl.at[pl.ds(base, N1)], dcl, dsem),
        ]
        for cp in cps:
            cp.start()
        for cp in cps:
            cp.wait()

        iota = lax.iota(jnp.int32, 16)
        zeros16i = jnp.zeros((16,), jnp.int32)
        zeros16f = jnp.zeros((16,), jnp.float32)

        for t in range(NC // 16):
            counts[pl.ds(t * 16, 16)] = zeros16i

        def body1(q, c):
            b16 = q * 16
            cls = dcl[pl.ds(b16, 16)].astype(jnp.int32)
            cnt, is_last = plsc.scan_count(cls)
            rank = cnt - 1
            ccl[pl.ds(b16, 16)] = cls
            rnk[pl.ds(b16, 16)] = rank
            lst[pl.ds(b16, 16)] = jnp.where(is_last,
                                            jnp.full((16,), 1, jnp.int32),
                                            jnp.zeros((16,), jnp.int32))
            plsc.addupdate_scatter(counts, [cls], cnt, mask=is_last)
            return c

        lax.fori_loop(0, NCH, body1, 0)

        run = jnp.int32(0)
        for t in range(NC // 16):
            v = counts[pl.ds(t * 16, 16)]
            cs = plsc.cumsum(v)
            ex = cs - v + run
            bases[pl.ds(t * 16, 16)] = ex
            wbase[pl.ds(t * 16, 16)] = ex
            run = run + jnp.max(cs)

        def body3(q, c):
            b16 = q * 16
            cls = ccl[pl.ds(b16, 16)]
            rank = rnk[pl.ds(b16, 16)]
            is_last = lst[pl.ds(b16, 16)] == 1
            pos = plsc.load_gather(wbase, [cls]) + rank
            plsc.store_scatter(bsc, [pos], dsc[pl.ds(b16, 16)])
            plsc.store_scatter(by1, [pos], dy1[pl.ds(b16, 16)])
            plsc.store_scatter(bx1, [pos], dx1[pl.ds(b16, 16)])
            plsc.store_scatter(by2, [pos], dy2[pl.ds(b16, 16)])
            plsc.store_scatter(bx2, [pos], dx2[pl.ds(b16, 16)])
            plsc.store_scatter(bcl, [pos], dcl[pl.ds(b16, 16)])
            plsc.store_scatter(bidx, [pos], b16 + iota)
            plsc.addupdate_scatter(wbase, [cls], rank + 1, mask=is_last)
            return c

        lax.fori_loop(0, NCH, body3, 0)

        def body4(q, c):
            b16 = q * 16
            s = bsc[pl.ds(b16, 16)]
            mc = jnp.max(s)
            bi = bidx[pl.ds(b16, 16)]
            tc = jnp.min(jnp.where(s == mc, bi, BIG))
            lane0 = iota == 0
            qv = jnp.full((16,), q, jnp.int32)
            plsc.store_scatter(cmax, [qv], jnp.full((16,), mc, jnp.float32),
                               mask=lane0)
            plsc.store_scatter(ctie, [qv], jnp.full((16,), tc, jnp.int32),
                               mask=lane0)
            return c

        lax.fori_loop(0, NCH, body4, 0)

        def global_max():
            c0 = cmax[pl.ds(0, 16)]
            c1 = cmax[pl.ds(16, 16)]
            c2 = cmax[pl.ds(32, 16)]
            c3 = cmax[pl.ds(48, 16)]
            return jnp.max(jnp.maximum(jnp.maximum(c0, c1),
                                       jnp.maximum(c2, c3)))

        def wcond(st):
            k, m = st
            return (k < K) & (m > 0.0)

        def wbody(st):
            k, m = st
            c0 = cmax[pl.ds(0, 16)]
            c1 = cmax[pl.ds(16, 16)]
            c2 = cmax[pl.ds(32, 16)]
            c3 = cmax[pl.ds(48, 16)]
            t0 = ctie[pl.ds(0, 16)]
            t1 = ctie[pl.ds(16, 16)]
            t2 = ctie[pl.ds(32, 16)]
            t3 = ctie[pl.ds(48, 16)]
            big = jnp.full((16,), BIG, jnp.int32)
            a0 = jnp.where(c0 == m, t0, big)
            a1 = jnp.where(c1 == m, t1, big)
            a2 = jnp.where(c2 == m, t2, big)
            a3 = jnp.where(c3 == m, t3, big)
            tmin = jnp.min(jnp.minimum(jnp.minimum(a0, a1),
                                       jnp.minimum(a2, a3)))
            q0 = jnp.where((c0 == m) & (t0 == tmin), iota, big)
            q1 = jnp.where((c1 == m) & (t1 == tmin), iota + 16, big)
            q2 = jnp.where((c2 == m) & (t2 == tmin), iota + 32, big)
            q3 = jnp.where((c3 == m) & (t3 == tmin), iota + 48, big)
            qs = jnp.min(jnp.minimum(jnp.minimum(q0, q1),
                                     jnp.minimum(q2, q3)))
            qs = jnp.clip(qs, 0, NCH - 1)
            b16 = qs * 16
            s = bsc[pl.ds(b16, 16)]
            bi = bidx[pl.ds(b16, 16)]
            ps = jnp.min(jnp.where((s == m) & (bi == tmin), iota, 15))
            g = b16 + ps
            gv = jnp.full((16,), g, jnp.int32)
            lane0 = iota == 0
            vy1 = plsc.load_gather(by1, [gv])
            vx1 = plsc.load_gather(bx1, [gv])
            vy2 = plsc.load_gather(by2, [gv])
            vx2 = plsc.load_gather(bx2, [gv])
            vcl = plsc.load_gather(bcl, [gv])
            soff = vcl * 10.0
            sny1 = vy1 + soff
            snx1 = vx1 + soff
            sny2 = vy2 + soff
            snx2 = vx2 + soff
            sarea = (sny2 - sny1) * (snx2 - snx1)
            out16 = jnp.where(iota == 0, vy1, 0.0)
            out16 = jnp.where(iota == 1, vx1, out16)
            out16 = jnp.where(iota == 2, vy2, out16)
            out16 = jnp.where(iota == 3, vx2, out16)
            out16 = jnp.where(iota == 4, vcl, out16)
            out16 = jnp.where(iota == 5, jnp.full((16,), m), out16)
            obuf[pl.ds(k * 16, 16)] = out16
            civ = jnp.clip(vcl.astype(jnp.int32), 0, NC - 1)
            a = jnp.max(plsc.load_gather(bases, [civ]))
            bnd = a + jnp.max(plsc.load_gather(counts, [civ]))
            plsc.store_scatter(bsc, [gv],
                               jnp.full((16,), -1.0, jnp.float32),
                               mask=lane0)
            qa = jnp.clip(lax.div(a, 16), 0, NCH - 1)
            qb = jnp.clip(lax.div(bnd + 15, 16), 0, NCH)

            def sbody(qq, c):
                o = qq * 16
                gi = o + iota
                sv = bsc[pl.ds(o, 16)]
                voff = bcl[pl.ds(o, 16)] * 10.0
                vny1 = by1[pl.ds(o, 16)] + voff
                vnx1 = bx1[pl.ds(o, 16)] + voff
                vny2 = by2[pl.ds(o, 16)] + voff
                vnx2 = bx2[pl.ds(o, 16)] + voff
                varea = (vny2 - vny1) * (vnx2 - vnx1)
                yy1 = jnp.maximum(vny1, sny1)
                xx1 = jnp.maximum(vnx1, snx1)
                yy2 = jnp.minimum(vny2, sny2)
                xx2 = jnp.minimum(vnx2, snx2)
                inter = (jnp.maximum(yy2 - yy1, 0.0)
                         * jnp.maximum(xx2 - xx1, 0.0))
                union = sarea + varea - inter
                iou = inter / jnp.maximum(union, 1e-8)
                kill = ((gi >= a) & (gi < bnd) & (gi != g)
                        & (iou > NMS_THR))
                s2 = jnp.where(kill, -1.0, sv)
                bsc[pl.ds(o, 16)] = s2
                mc = jnp.max(s2)
                tc = jnp.min(jnp.where(s2 == mc, bidx[pl.ds(o, 16)], BIG))
                qv = jnp.full((16,), qq, jnp.int32)
                plsc.store_scatter(cmax, [qv],
                                   jnp.full((16,), mc, jnp.float32),
                                   mask=lane0)
                plsc.store_scatter(ctie, [qv],
                                   jnp.full((16,), tc, jnp.int32),
                                   mask=lane0)
                return c

            sbody(qa, 0)

            @pl.when(qa + 1 < qb)
            def _():
                sbody(qa + 1, 0)

            lax.fori_loop(qa + 2, qb, sbody, 0)
            return (k + 1, global_max())

        kf, _ = lax.while_loop(wcond, wbody, (jnp.int32(0), global_max()))

        def bodyz(q, c):
            obuf[pl.ds(q * 16, 16)] = zeros16f
            return c

        lax.fori_loop(kf, K, bodyz, 0)
        pltpu.sync_copy(obuf, out_hbm.at[wid])


def _sc_nms(data):
    mesh = plsc.VectorSubcoreMesh(core_axis_name="c", subcore_axis_name="s")
    f32 = jnp.float32
    i32 = jnp.int32
    run = functools.partial(
        pl.kernel,
        mesh=mesh,
        compiler_params=pltpu.CompilerParams(needs_layout_passes=False),
        out_type=jax.ShapeDtypeStruct((B, K * 16), f32),
        scratch_types=[
            pltpu.VMEM((N1,), f32),
            pltpu.VMEM((N1,), f32),
            pltpu.VMEM((N1,), f32),
            pltpu.VMEM((N1,), f32),
            pltpu.VMEM((N1,), f32),
            pltpu.VMEM((N1,), f32),
            pltpu.VMEM((N1,), i32),
            pltpu.VMEM((N1,), i32),
            pltpu.VMEM((N1,), i32),
            pltpu.VMEM((N1,), f32),
            pltpu.VMEM((N1,), f32),
            pltpu.VMEM((N1,), f32),
            pltpu.VMEM((N1,), f32),
            pltpu.VMEM((N1,), f32),
            pltpu.VMEM((N1,), f32),
            pltpu.VMEM((N1,), i32),
            pltpu.VMEM((NC,), i32),
            pltpu.VMEM((NC,), i32),
            pltpu.VMEM((NC,), i32),
            pltpu.VMEM((NCH,), f32),
            pltpu.VMEM((NCH,), i32),
            pltpu.VMEM((K * 16,), f32),
            pltpu.SemaphoreType.DMA,
        ],
    )(_sc_nms_body)
    return run(*data)


def kernel(rois, mrcnn_class, mrcnn_bbox):
    rois_t = jnp.transpose(rois, (0, 2, 1))
    probs_t = jnp.transpose(mrcnn_class, (2, 0, 1))
    flat_t = jnp.transpose(mrcnn_bbox, (0, 2, 3, 1))
    data = pl.pallas_call(
        _dense_body,
        out_shape=[jax.ShapeDtypeStruct((B * N1,), jnp.float32)] * 6,
    )(rois_t, probs_t, flat_t)
    out = _sc_nms(data)
    det = out.reshape(B, K, 16)[:, :, :6]
    return det


# speedup vs baseline: 2.0075x; 1.0537x over previous
import functools

import jax
import jax.numpy as jnp
from jax import lax
from jax.experimental import pallas as pl
from jax.experimental.pallas import tpu as pltpu
from jax.experimental.pallas import tpu_sc as plsc

B = 8
N = 1000
C = 81
K = 100
MIN_CONF = 0.7
NMS_THR = 0.3
N1 = 1024
NCH = N1 // 16
NC = 96
BIG = 2**30


def _dense_body(rois_ref, probs_ref, flat_ref, osc_ref, oy1_ref, ox1_ref,
                oy2_ref, ox2_ref, ocl_ref):
    for b in range(B):
        _dense_one(rois_ref[b], probs_ref[:, b, :], flat_ref[b],
                   b, osc_ref, oy1_ref, ox1_ref, oy2_ref, ox2_ref, ocl_ref)


def _dense_one(rois, probs, flat4, b, osc_ref, oy1_ref, ox1_ref,
               oy2_ref, ox2_ref, ocl_ref):
    m = jnp.max(probs, axis=0, keepdims=True)
    iota_c = lax.broadcasted_iota(jnp.int32, (C, N), 0)
    cid = jnp.min(jnp.where(probs == m, iota_c, C), axis=0, keepdims=True)

    maskf = (iota_c == cid).astype(jnp.float32)
    s = flat4 * maskf[:, None, :]
    while s.shape[0] > 1:
        half = s.shape[0] // 2
        even = s[:half] + s[half:2 * half]
        if s.shape[0] % 2:
            even = jnp.concatenate(
                [even[:1] + s[2 * half:], even[1:]], axis=0)
        s = even
    dsel = s[0]
    dy = dsel[0:1] * 0.1
    dx = dsel[1:2] * 0.1
    dh = dsel[2:3] * 0.2
    dw = dsel[3:4] * 0.2

    ry1 = rois[0:1]
    rx1 = rois[1:2]
    ry2 = rois[2:3]
    rx2 = rois[3:4]
    h = ry2 - ry1
    w = rx2 - rx1
    cy = ry1 + 0.5 * h
    cx = rx1 + 0.5 * w
    cy = cy + dy * h
    cx = cx + dx * w
    h = h * jnp.exp(dh)
    w = w * jnp.exp(dw)
    y1 = jnp.clip(cy - 0.5 * h, 0.0, 1.0)
    x1 = jnp.clip(cx - 0.5 * w, 0.0, 1.0)
    y2 = jnp.clip(cy + 0.5 * h, 0.0, 1.0)
    x2 = jnp.clip(cx + 0.5 * w, 0.0, 1.0)

    valid = (cid > 0) & (m >= MIN_CONF)
    sc = jnp.where(valid, m, -1.0)
    clsf = cid.astype(jnp.float32)

    padn = jnp.full((N1 - N,), -1.0, jnp.float32)
    padz = jnp.zeros((N1 - N,), jnp.float32)
    o = b * N1
    osc_ref[pl.ds(o, N1)] = jnp.concatenate([jnp.reshape(sc, (N,)), padn])
    oy1_ref[pl.ds(o, N1)] = jnp.concatenate([jnp.reshape(y1, (N,)), padz])
    ox1_ref[pl.ds(o, N1)] = jnp.concatenate([jnp.reshape(x1, (N,)), padz])
    oy2_ref[pl.ds(o, N1)] = jnp.concatenate([jnp.reshape(y2, (N,)), padz])
    ox2_ref[pl.ds(o, N1)] = jnp.concatenate([jnp.reshape(x2, (N,)), padz])
    ocl_ref[pl.ds(o, N1)] = jnp.concatenate([jnp.reshape(clsf, (N,)), padz])


def _sc_nms_body(hsc, hy1, hx1, hy2, hx2, hcl, out_hbm,
                 dsc, dy1, dx1, dy2, dx2, dcl,
                 ccl, rnk, lst,
                 bsc, by1, bx1, by2, bx2, bcl, bidx,
                 counts, bases, wbase, cmax, ctie, obuf, dsem):
    wid = lax.axis_index("s") * 2 + lax.axis_index("c")

    @pl.when(wid < B)
    def _():
        base = wid * N1
        cps = [
            pltpu.make_async_copy(hsc.at[pl.ds(base, N1)], dsc, dsem),
            pltpu.make_async_copy(hy1.at[pl.ds(base, N1)], dy1, dsem),
            pltpu.make_async_copy(hx1.at[pl.ds(base, N1)], dx1, dsem),
            pltpu.make_async_copy(hy2.at[pl.ds(base, N1)], dy2, dsem),
            pltpu.make_async_copy(hx2.at[pl.ds(base, N1)], dx2, dsem),
            pltpu.make_async_copy(hcl.at[pl.ds(base, N1)], dcl, dsem),
        ]
        for cp in cps:
            cp.start()
        for cp in cps:
            cp.wait()

        iota = lax.iota(jnp.int32, 16)
        zeros16i = jnp.zeros((16,), jnp.int32)
        zeros16f = jnp.zeros((16,), jnp.float32)

        for t in range(NC // 16):
            counts[pl.ds(t * 16, 16)] = zeros16i

        def body1(q, c):
            b16 = q * 16
            cls = dcl[pl.ds(b16, 16)].astype(jnp.int32)
            cnt, is_last = plsc.scan_count(cls)
            rank = cnt - 1
            ccl[pl.ds(b16, 16)] = cls
            rnk[pl.ds(b16, 16)] = rank
            lst[pl.ds(b16, 16)] = jnp.where(is_last,
                                            jnp.full((16,), 1, jnp.int32),
                                            jnp.zeros((16,), jnp.int32))
            plsc.addupdate_scatter(counts, [cls], cnt, mask=is_last)
            return c

        lax.fori_loop(0, NCH, body1, 0)

        run = jnp.int32(0)
        for t in range(NC // 16):
            v = counts[pl.ds(t * 16, 16)]
            cs = plsc.cumsum(v)
            ex = cs - v + run
            bases[pl.ds(t * 16, 16)] = ex
            wbase[pl.ds(t * 16, 16)] = ex
            run = run + jnp.max(cs)

        def body3(q, c):
            b16 = q * 16
            cls = ccl[pl.ds(b16, 16)]
            rank = rnk[pl.ds(b16, 16)]
            is_last = lst[pl.ds(b16, 16)] == 1
            pos = plsc.load_gather(wbase, [cls]) + rank
            plsc.store_scatter(bsc, [pos], dsc[pl.ds(b16, 16)])
            plsc.store_scatter(by1, [pos], dy1[pl.ds(b16, 16)])
            plsc.store_scatter(bx1, [pos], dx1[pl.ds(b16, 16)])
            plsc.store_scatter(by2, [pos], dy2[pl.ds(b16, 16)])
            plsc.store_scatter(bx2, [pos], dx2[pl.ds(b16, 16)])
            plsc.store_scatter(bcl, [pos], dcl[pl.ds(b16, 16)])
            plsc.store_scatter(bidx, [pos], b16 + iota)
            plsc.addupdate_scatter(wbase, [cls], rank + 1, mask=is_last)
            return c

        lax.fori_loop(0, NCH, body3, 0)

        def body4(q, c):
            b16 = q * 16
            s = bsc[pl.ds(b16, 16)]
            mc = jnp.max(s)
            bi = bidx[pl.ds(b16, 16)]
            tc = jnp.min(jnp.where(s == mc, bi, BIG))
            lane0 = iota == 0
            qv = jnp.full((16,), q, jnp.int32)
            plsc.store_scatter(cmax, [qv], jnp.full((16,), mc, jnp.float32),
                               mask=lane0)
            plsc.store_scatter(ctie, [qv], jnp.full((16,), tc, jnp.int32),
                               mask=lane0)
            return c

        lax.fori_loop(0, NCH, body4, 0)

        def gmax(c0, c1, c2, c3):
            return jnp.max(jnp.maximum(jnp.maximum(c0, c1),
                                       jnp.maximum(c2, c3)))

        def wcond(st):
            return (st[0] < K) & (st[1] > 0.0)

        def wbody(st):
            k, m, c0, c1, c2, c3, t0, t1, t2, t3 = st
            big = jnp.full((16,), BIG, jnp.int32)
            a0 = jnp.where(c0 == m, t0, big)
            a1 = jnp.where(c1 == m, t1, big)
            a2 = jnp.where(c2 == m, t2, big)
            a3 = jnp.where(c3 == m, t3, big)
            tmin = jnp.min(jnp.minimum(jnp.minimum(a0, a1),
                                       jnp.minimum(a2, a3)))
            q0 = jnp.where((c0 == m) & (t0 == tmin), iota, big)
            q1 = jnp.where((c1 == m) & (t1 == tmin), iota + 16, big)
            q2 = jnp.where((c2 == m) & (t2 == tmin), iota + 32, big)
            q3 = jnp.where((c3 == m) & (t3 == tmin), iota + 48, big)
            qs = jnp.min(jnp.minimum(jnp.minimum(q0, q1),
                                     jnp.minimum(q2, q3)))
            qs = jnp.clip(qs, 0, NCH - 1)
            b16 = qs * 16
            s = bsc[pl.ds(b16, 16)]
            bi = bidx[pl.ds(b16, 16)]
            ps = jnp.min(jnp.where((s == m) & (bi == tmin), iota, 15))
            g = b16 + ps
            gv = jnp.full((16,), g, jnp.int32)
            lane0 = iota == 0
            vy1 = plsc.load_gather(by1, [gv])
            vx1 = plsc.load_gather(bx1, [gv])
            vy2 = plsc.load_gather(by2, [gv])
            vx2 = plsc.load_gather(bx2, [gv])
            vcl = plsc.load_gather(bcl, [gv])
            soff = vcl * 10.0
            sny1 = vy1 + soff
            snx1 = vx1 + soff
            sny2 = vy2 + soff
            snx2 = vx2 + soff
            sarea = (sny2 - sny1) * (snx2 - snx1)
            out16 = jnp.where(iota == 0, vy1, 0.0)
            out16 = jnp.where(iota == 1, vx1, out16)
            out16 = jnp.where(iota == 2, vy2, out16)
            out16 = jnp.where(iota == 3, vx2, out16)
            out16 = jnp.where(iota == 4, vcl, out16)
            out16 = jnp.where(iota == 5, jnp.full((16,), m), out16)
            obuf[pl.ds(k * 16, 16)] = out16
            civ = jnp.clip(vcl.astype(jnp.int32), 0, NC - 1)
            a = jnp.max(plsc.load_gather(bases, [civ]))
            bnd = a + jnp.max(plsc.load_gather(counts, [civ]))
            plsc.store_scatter(bsc, [gv],
                               jnp.full((16,), -1.0, jnp.float32),
                               mask=lane0)
            qa = jnp.clip(lax.div(a, 16), 0, NCH - 1)
            qb = jnp.clip(lax.div(bnd + 15, 16), 0, NCH)

            def sbody(qq, tabs):
                sc0, sc1, sc2, sc3, st0, st1, st2, st3 = tabs
                o = qq * 16
                gi = o + iota
                sv = bsc[pl.ds(o, 16)]
                voff = bcl[pl.ds(o, 16)] * 10.0
                vny1 = by1[pl.ds(o, 16)] + voff
                vnx1 = bx1[pl.ds(o, 16)] + voff
                vny2 = by2[pl.ds(o, 16)] + voff
                vnx2 = bx2[pl.ds(o, 16)] + voff
                varea = (vny2 - vny1) * (vnx2 - vnx1)
                yy1 = jnp.maximum(vny1, sny1)
                xx1 = jnp.maximum(vnx1, snx1)
                yy2 = jnp.minimum(vny2, sny2)
                xx2 = jnp.minimum(vnx2, snx2)
                inter = (jnp.maximum(yy2 - yy1, 0.0)
                         * jnp.maximum(xx2 - xx1, 0.0))
                union = sarea + varea - inter
                iou = inter / jnp.maximum(union, 1e-8)
                kill = ((gi >= a) & (gi < bnd) & (gi != g)
                        & (iou > NMS_THR))
                s2 = jnp.where(kill, -1.0, sv)
                bsc[pl.ds(o, 16)] = s2
                mc = jnp.max(s2)
                tc = jnp.min(jnp.where(s2 == mc, bidx[pl.ds(o, 16)], BIG))
                h0 = iota == qq
                h1 = iota + 16 == qq
                h2 = iota + 32 == qq
                h3 = iota + 48 == qq
                sc0 = jnp.where(h0, mc, sc0)
                sc1 = jnp.where(h1, mc, sc1)
                sc2 = jnp.where(h2, mc, sc2)
                sc3 = jnp.where(h3, mc, sc3)
                st0 = jnp.where(h0, tc, st0)
                st1 = jnp.where(h1, tc, st1)
                st2 = jnp.where(h2, tc, st2)
                st3 = jnp.where(h3, tc, st3)
                return (sc0, sc1, sc2, sc3, st0, st1, st2, st3)

            tabs = (c0, c1, c2, c3, t0, t1, t2, t3)
            tabs = sbody(qa, tabs)
            tabs = sbody(jnp.minimum(qa + 1, NCH - 1), tabs)
            tabs = lax.fori_loop(qa + 2, qb, sbody, tabs)
            c0, c1, c2, c3, t0, t1, t2, t3 = tabs
            return (k + 1, gmax(c0, c1, c2, c3),
                    c0, c1, c2, c3, t0, t1, t2, t3)

        i0 = cmax[pl.ds(0, 16)]
        i1 = cmax[pl.ds(16, 16)]
        i2 = cmax[pl.ds(32, 16)]
        i3 = cmax[pl.ds(48, 16)]
        j0 = ctie[pl.ds(0, 16)]
        j1 = ctie[pl.ds(16, 16)]
        j2 = ctie[pl.ds(32, 16)]
        j3 = ctie[pl.ds(48, 16)]
        st = lax.while_loop(
            wcond, wbody,
            (jnp.int32(0), gmax(i0, i1, i2, i3),
             i0, i1, i2, i3, j0, j1, j2, j3))
        kf = st[0]

        def bodyz(q, c):
            obuf[pl.ds(q * 16, 16)] = zeros16f
            return c

        lax.fori_loop(kf, K, bodyz, 0)
        pltpu.sync_copy(obuf, out_hbm.at[wid])


def _sc_nms(data):
    mesh = plsc.VectorSubcoreMesh(core_axis_name="c", subcore_axis_name="s")
    f32 = jnp.float32
    i32 = jnp.int32
    run = functools.partial(
        pl.kernel,
        mesh=mesh,
        compiler_params=pltpu.CompilerParams(needs_layout_passes=False),
        out_type=jax.ShapeDtypeStruct((B, K * 16), f32),
        scratch_types=[
            pltpu.VMEM((N1,), f32),
            pltpu.VMEM((N1,), f32),
            pltpu.VMEM((N1,), f32),
            pltpu.VMEM((N1,), f32),
            pltpu.VMEM((N1,), f32),
            pltpu.VMEM((N1,), f32),
            pltpu.VMEM((N1,), i32),
            pltpu.VMEM((N1,), i32),
            pltpu.VMEM((N1,), i32),
            pltpu.VMEM((N1,), f32),
            pltpu.VMEM((N1,), f32),
            pltpu.VMEM((N1,), f32),
            pltpu.VMEM((N1,), f32),
            pltpu.VMEM((N1,), f32),
            pltpu.VMEM((N1,), f32),
            pltpu.VMEM((N1,), i32),
            pltpu.VMEM((NC,), i32),
            pltpu.VMEM((NC,), i32),
            pltpu.VMEM((NC,), i32),
            pltpu.VMEM((NCH,), f32),
            pltpu.VMEM((NCH,), i32),
            pltpu.VMEM((K * 16,), f32),
            pltpu.SemaphoreType.DMA,
        ],
    )(_sc_nms_body)
    return run(*data)


def kernel(rois, mrcnn_class, mrcnn_bbox):
    rois_t = jnp.transpose(rois, (0, 2, 1))
    probs_t = jnp.transpose(mrcnn_class, (2, 0, 1))
    flat_t = jnp.transpose(mrcnn_bbox, (0, 2, 3, 1))
    data = pl.pallas_call(
        _dense_body,
        out_shape=[jax.ShapeDtypeStruct((B * N1,), jnp.float32)] * 6,
    )(rois_t, probs_t, flat_t)
    out = _sc_nms(data)
    det = out.reshape(B, K, 16)[:, :, :6]
    return det
